# Initial kernel scaffold; baseline (speedup 1.0000x reference)
import jax
import jax.numpy as jnp
from jax.experimental import pallas as pl


def kernel(input_feature, edge_index, edge_weight, weight, bias, w1, b):
    raise NotImplementedError("write your pallas kernel here")


# speedup vs baseline: 15.9243x; 15.9243x over previous
import jax
import jax.numpy as jnp
from jax import lax
from jax.experimental import pallas as pl
from jax.experimental.pallas import tpu as pltpu
from jax.experimental.pallas import tpu_sc as plsc

C = 1.0
KEEP = 5000
N = 10000
NPAD = 10240
D = 128
E = 320000
MIN_NORM = 1e-15
EPS = 1e-5

NC = 2
NS = 16
NW = NC * NS
EPT = E // NW
NODES_PER_SUB = NPAD // NS
ROWS_OUT = 5120
ROWS_PER_SUB = ROWS_OUT // NW


def _k1_body(x_ref, wt_ref, bias_ref, norm_in_ref, s_ref, norm_ref, f_ref):
    x = x_ref[...]
    norm = norm_in_ref[...]
    ncl = jnp.maximum(norm, MIN_NORM)
    scaled = jnp.minimum(ncl, 1.0 - EPS)
    arct = 0.5 * (jnp.log1p(scaled) - jnp.log1p(-scaled))
    f = arct / ncl
    xt = f * x
    dot = jnp.dot(xt, wt_ref[...],
                  preferred_element_type=jnp.float32)
    hb = bias_ref[0, 0]
    hbn = jnp.maximum(jnp.sqrt(hb * hb), MIN_NORM)
    e = jnp.tanh(hbn) * hb / hbn
    maxn = 1.0 - EPS
    en = jnp.maximum(jnp.sqrt(e * e), MIN_NORM)
    h = jnp.where(en > maxn, e / en * maxn, e)
    x2 = dot * dot
    y2 = h * h
    xy = dot * h
    num = (1.0 + 2.0 * xy + y2) * dot + (1.0 - x2) * h
    den = 1.0 + 2.0 * xy + x2 * y2
    m = num / jnp.maximum(den, MIN_NORM)
    mn = jnp.maximum(jnp.sqrt(m * m), MIN_NORM)
    s = jnp.where(mn > maxn, m / mn * maxn, m)
    s_ref[...] = s
    norm_ref[...] = norm
    f_ref[...] = f


def _k2_body(src_hbm, dst_hbm, ew_hbm, s_hbm, agg_hbm,
             src_v, dst_v, ew_v, sval_v, val_v, zero_v, agg_sh, sem):
    cid = lax.axis_index("c")
    sid = lax.axis_index("s")
    wid = cid * NS + sid
    def zb(i, _):
        zero_v[pl.ds(i * 16, 16)] = jnp.zeros((16,), jnp.float32)
        return 0
    lax.fori_loop(0, NODES_PER_SUB // 16, zb, 0)
    pltpu.sync_copy(zero_v, agg_sh.at[pl.ds(sid * NODES_PER_SUB, NODES_PER_SUB)])
    pltpu.sync_copy(src_hbm.at[wid], src_v)
    pltpu.sync_copy(dst_hbm.at[wid], dst_v)
    pltpu.sync_copy(ew_hbm.at[wid], ew_v)
    pltpu.async_copy(s_hbm.at[src_v], sval_v, sem).wait()
    def body(i, _):
        sl = pl.ds(i * 16, 16)
        val_v[sl] = sval_v[sl] * ew_v[sl]
        return 0
    lax.fori_loop(0, EPT // 16, body, 0)
    plsc.subcore_barrier()
    pltpu.sync_copy(val_v, agg_sh.at[dst_v], add=True)
    plsc.subcore_barrier()
    pltpu.sync_copy(agg_sh.at[pl.ds(sid * NODES_PER_SUB, NODES_PER_SUB)],
                    agg_hbm.at[cid, pl.ds(sid * NODES_PER_SUB, NODES_PER_SUB)])


def _k3_body(agg_ref, norm_ref, f_ref, w1_ref, b_ref,
             attn_ref, p_ref, g_ref):
    R = NPAD // 128
    agg = agg_ref[0:R, :] + agg_ref[R:2 * R, :]
    un = jnp.maximum(jnp.sqrt(agg * agg), MIN_NORM)
    e0 = jnp.tanh(un) * agg / un
    maxn = 1.0 - EPS
    e0n = jnp.maximum(jnp.sqrt(e0 * e0), MIN_NORM)
    e0p = jnp.where(e0n > maxn, e0 / e0n * maxn, e0)
    at = jnp.tanh(e0p)
    w1 = w1_ref[0, 0]
    b = b_ref[0, 0]
    attn = w1 * at + (1.0 - w1) * (1.0 - norm_ref[...]) + b
    attn_ref[...] = attn
    rid = lax.broadcasted_iota(jnp.int32, (R, 128), 0)
    cidx = lax.broadcasted_iota(jnp.int32, (R, 128), 1)
    nid = rid * 128 + cidx
    valid = nid < N
    attn_eff = jnp.where(valid, attn, -jnp.inf)
    bu = lax.bitcast_convert_type(attn_eff, jnp.uint32)
    sign = bu >> jnp.uint32(31)
    flip = jnp.where(sign == jnp.uint32(1),
                     jnp.uint32(0xFFFFFFFF), jnp.uint32(0x80000000))
    u = bu ^ flip
    kf32 = jnp.float32(KEEP)
    def bit_body(j, t):
        cand = t | (jnp.uint32(1) << jnp.uint32(31 - j))
        cnt = jnp.sum(jnp.where(u >= cand, 1.0, 0.0))
        return jnp.where(cnt >= kf32, cand, t)
    thr = lax.fori_loop(0, 32, bit_body, jnp.uint32(0))
    gt = u > thr
    eq = u == thr
    cnt_gt = jnp.sum(jnp.where(gt, 1.0, 0.0))
    l_ids = lax.broadcasted_iota(jnp.int32, (128, 128), 0)
    c_ids = lax.broadcasted_iota(jnp.int32, (128, 128), 1)
    tri = (l_ids <= c_ids).astype(jnp.float32)
    r_ids = lax.broadcasted_iota(jnp.int32, (R, R), 0)
    q_ids = lax.broadcasted_iota(jnp.int32, (R, R), 1)
    strict = (q_ids < r_ids).astype(jnp.float32)

    def excl_cumsum(mf):
        incl = jnp.dot(mf, tri, preferred_element_type=jnp.float32)
        row_tot = jnp.sum(mf, axis=1, keepdims=True)
        off = jnp.dot(strict, row_tot, preferred_element_type=jnp.float32)
        return incl - mf + off

    eqf = eq.astype(jnp.float32)
    eq_rank = excl_cumsum(eqf)
    keep = gt | (eq & (eq_rank < (kf32 - cnt_gt)))
    kpf = keep.astype(jnp.float32)
    kc = excl_cumsum(kpf)
    nidf = nid.astype(jnp.float32)
    pos = jnp.where(keep, kc, kf32 + (nidf - kc))
    p_ref[...] = pos.astype(jnp.int32)
    g_ref[...] = jnp.where(valid, f_ref[...] * attn, 0.0)


def _k4_body(p_hbm, g_hbm, x_hbm, out_hbm,
             p_v, ids_v, idx_v, gk_v, rows_v, kidx_sh, sem):
    cid = lax.axis_index("c")
    sid = lax.axis_index("s")
    pltpu.sync_copy(p_hbm.at[sid], p_v)
    def ib(i, _):
        ids_v[pl.ds(i * 16, 16)] = (sid * NODES_PER_SUB + i * 16
                                    + lax.broadcasted_iota(jnp.int32, (16,), 0))
        return 0
    lax.fori_loop(0, NODES_PER_SUB // 16, ib, 0)
    pltpu.sync_copy(ids_v, kidx_sh.at[p_v])
    plsc.subcore_barrier()
    wid = cid * NS + sid
    base = wid * ROWS_PER_SUB
    pltpu.sync_copy(kidx_sh.at[pl.ds(base, ROWS_PER_SUB)], idx_v)
    pltpu.async_copy(g_hbm.at[idx_v], gk_v, sem).wait()
    pltpu.async_copy(x_hbm.at[idx_v], rows_v, sem).wait()
    def rb(cchunk, _):
        gv = gk_v[pl.ds(cchunk * 16, 16)]
        for r in range(16):
            gr = gv[r]
            row = cchunk * 16 + r
            for l in range(8):
                sl = pl.ds(l * 16, 16)
                rows_v[row, sl] = rows_v[row, sl] * gr
        return 0
    lax.fori_loop(0, ROWS_PER_SUB // 16, rb, 0)
    pltpu.sync_copy(rows_v, out_hbm.at[pl.ds(base, ROWS_PER_SUB)])


def _sc_mesh():
    return plsc.VectorSubcoreMesh(core_axis_name="c", subcore_axis_name="s",
                                  num_cores=NC, num_subcores=NS)


@jax.jit
def kernel(input_feature, edge_index, edge_weight, weight, bias, w1, b):
    xp = jnp.pad(input_feature, ((0, NPAD - N), (0, 0)))
    bias11 = bias.reshape(1, 1).astype(jnp.float32)
    w1a = jnp.asarray(w1, jnp.float32).reshape(1, 1)
    ba = jnp.asarray(b, jnp.float32).reshape(1, 1)
    norm_in = jnp.linalg.norm(xp, axis=-1, keepdims=True)

    s_col, norm_col, f_col = pl.pallas_call(
        _k1_body,
        out_shape=[jax.ShapeDtypeStruct((NPAD, 1), jnp.float32)] * 3,
    )(xp, weight, bias11, norm_in)

    src2 = edge_index[0].reshape(NW, EPT)
    dst2 = edge_index[1].reshape(NW, EPT)
    ew2 = edge_weight.reshape(NW, EPT)
    s_flat = s_col.reshape(NPAD)

    k2 = pl.kernel(
        _k2_body,
        out_type=jax.ShapeDtypeStruct((NC, NPAD), jnp.float32),
        mesh=_sc_mesh(),
        scratch_types=[
            pltpu.VMEM((EPT,), jnp.int32),
            pltpu.VMEM((EPT,), jnp.int32),
            pltpu.VMEM((EPT,), jnp.float32),
            pltpu.VMEM((EPT,), jnp.float32),
            pltpu.VMEM((EPT,), jnp.float32),
            pltpu.VMEM((NODES_PER_SUB,), jnp.float32),
            pltpu.VMEM_SHARED((NPAD,), jnp.float32),
            pltpu.SemaphoreType.DMA,
        ],
    )
    aggp = k2(src2, dst2, ew2, s_flat)

    attn2d, p2d, g2d = pl.pallas_call(
        _k3_body,
        out_shape=[
            jax.ShapeDtypeStruct((NPAD // 128, 128), jnp.float32),
            jax.ShapeDtypeStruct((NPAD // 128, 128), jnp.int32),
            jax.ShapeDtypeStruct((NPAD // 128, 128), jnp.float32),
        ],
    )(aggp.reshape(2 * (NPAD // 128), 128),
      norm_col.reshape(NPAD // 128, 128),
      f_col.reshape(NPAD // 128, 128), w1a, ba)

    attn_score = attn2d.reshape(NPAD)[:N]
    p16 = p2d.reshape(NS, NODES_PER_SUB)
    g_flat = g2d.reshape(NPAD)

    k4 = pl.kernel(
        _k4_body,
        out_type=jax.ShapeDtypeStruct((ROWS_OUT, D), jnp.float32),
        mesh=_sc_mesh(),
        scratch_types=[
            pltpu.VMEM((NODES_PER_SUB,), jnp.int32),
            pltpu.VMEM((NODES_PER_SUB,), jnp.int32),
            pltpu.VMEM((ROWS_PER_SUB,), jnp.int32),
            pltpu.VMEM((ROWS_PER_SUB,), jnp.float32),
            pltpu.VMEM((ROWS_PER_SUB, D), jnp.float32),
            pltpu.VMEM_SHARED((NPAD,), jnp.int32),
            pltpu.SemaphoreType.DMA,
        ],
    )
    hidden_pad = k4(p16, g_flat, input_feature)
    hidden = hidden_pad[:KEEP]
    return hidden, attn_score


# speedup vs baseline: 16.0113x; 1.0055x over previous
import jax
import jax.numpy as jnp
from jax import lax
from jax.experimental import pallas as pl
from jax.experimental.pallas import tpu as pltpu
from jax.experimental.pallas import tpu_sc as plsc

C = 1.0
KEEP = 5000
N = 10000
NPAD = 10240
D = 128
E = 320000
MIN_NORM = 1e-15
EPS = 1e-5

NC = 2
NS = 16
NW = NC * NS
EPT = E // NW
NODES_PER_SUB = NPAD // NS
ROWS_OUT = 5120
ROWS_PER_SUB = ROWS_OUT // NW


def _k1_body(x_ref, wt_ref, bias_ref, norm_in_ref, s_ref, norm_ref, f_ref):
    x = x_ref[...]
    norm = norm_in_ref[...]
    ncl = jnp.maximum(norm, MIN_NORM)
    scaled = jnp.minimum(ncl, 1.0 - EPS)
    arct = 0.5 * (jnp.log1p(scaled) - jnp.log1p(-scaled))
    f = arct / ncl
    xt = f * x
    dot = jnp.dot(xt, wt_ref[...],
                  preferred_element_type=jnp.float32)
    hb = bias_ref[0, 0]
    hbn = jnp.maximum(jnp.sqrt(hb * hb), MIN_NORM)
    e = jnp.tanh(hbn) * hb / hbn
    maxn = 1.0 - EPS
    en = jnp.maximum(jnp.sqrt(e * e), MIN_NORM)
    h = jnp.where(en > maxn, e / en * maxn, e)
    x2 = dot * dot
    y2 = h * h
    xy = dot * h
    num = (1.0 + 2.0 * xy + y2) * dot + (1.0 - x2) * h
    den = 1.0 + 2.0 * xy + x2 * y2
    m = num / jnp.maximum(den, MIN_NORM)
    mn = jnp.maximum(jnp.sqrt(m * m), MIN_NORM)
    s = jnp.where(mn > maxn, m / mn * maxn, m)
    s_ref[...] = s
    norm_ref[...] = norm
    f_ref[...] = f


def _k2_body(src_hbm, dst_hbm, ew_hbm, s_hbm, zeros_hbm, agg_hbm,
             src_v, dst_v, ew_v, sval_v, val_v, agg_sh,
             sem_s, sem_d, sem_e, sem_g, sem_z):
    cid = lax.axis_index("c")
    sid = lax.axis_index("s")
    wid = cid * NS + sid
    c_src = pltpu.async_copy(src_hbm.at[wid], src_v, sem_s)
    c_dst = pltpu.async_copy(dst_hbm.at[wid], dst_v, sem_d)
    c_ew = pltpu.async_copy(ew_hbm.at[wid], ew_v, sem_e)
    c_z = pltpu.async_copy(zeros_hbm.at[pl.ds(sid * NODES_PER_SUB, NODES_PER_SUB)],
                           agg_sh.at[pl.ds(sid * NODES_PER_SUB, NODES_PER_SUB)],
                           sem_z)
    c_src.wait()
    pltpu.async_copy(s_hbm.at[src_v], sval_v, sem_g).wait()
    c_ew.wait()
    def body(i, _):
        for u in range(5):
            sl = pl.ds(i * 80 + u * 16, 16)
            val_v[sl] = sval_v[sl] * ew_v[sl]
        return 0
    lax.fori_loop(0, EPT // 80, body, 0)
    c_dst.wait()
    c_z.wait()
    plsc.subcore_barrier()
    pltpu.sync_copy(val_v, agg_sh.at[dst_v], add=True)
    plsc.subcore_barrier()
    pltpu.sync_copy(agg_sh.at[pl.ds(sid * NODES_PER_SUB, NODES_PER_SUB)],
                    agg_hbm.at[cid, pl.ds(sid * NODES_PER_SUB, NODES_PER_SUB)])


def _k3_body(agg_ref, norm_ref, f_ref, w1_ref, b_ref,
             attn_ref, p_ref, g_ref):
    R = NPAD // 128
    agg = agg_ref[0:R, :] + agg_ref[R:2 * R, :]
    un = jnp.maximum(jnp.sqrt(agg * agg), MIN_NORM)
    e0 = jnp.tanh(un) * agg / un
    maxn = 1.0 - EPS
    e0n = jnp.maximum(jnp.sqrt(e0 * e0), MIN_NORM)
    e0p = jnp.where(e0n > maxn, e0 / e0n * maxn, e0)
    at = jnp.tanh(e0p)
    w1 = w1_ref[0, 0]
    b = b_ref[0, 0]
    attn = w1 * at + (1.0 - w1) * (1.0 - norm_ref[...]) + b
    attn_ref[...] = attn
    rid = lax.broadcasted_iota(jnp.int32, (R, 128), 0)
    cidx = lax.broadcasted_iota(jnp.int32, (R, 128), 1)
    nid = rid * 128 + cidx
    valid = nid < N
    attn_eff = jnp.where(valid, attn, -jnp.inf)
    bu = lax.bitcast_convert_type(attn_eff, jnp.uint32)
    sign = bu >> jnp.uint32(31)
    flip = jnp.where(sign == jnp.uint32(1),
                     jnp.uint32(0xFFFFFFFF), jnp.uint32(0x80000000))
    u = bu ^ flip
    kf32 = jnp.float32(KEEP)
    def bit_body(j, t):
        cand = t | (jnp.uint32(1) << jnp.uint32(31 - j))
        cnt = jnp.sum(jnp.where(u >= cand, 1.0, 0.0))
        return jnp.where(cnt >= kf32, cand, t)
    thr = lax.fori_loop(0, 32, bit_body, jnp.uint32(0))
    gt = u > thr
    eq = u == thr
    cnt_gt = jnp.sum(jnp.where(gt, 1.0, 0.0))
    l_ids = lax.broadcasted_iota(jnp.int32, (128, 128), 0)
    c_ids = lax.broadcasted_iota(jnp.int32, (128, 128), 1)
    tri = (l_ids <= c_ids).astype(jnp.float32)
    r_ids = lax.broadcasted_iota(jnp.int32, (R, R), 0)
    q_ids = lax.broadcasted_iota(jnp.int32, (R, R), 1)
    strict = (q_ids < r_ids).astype(jnp.float32)

    def excl_cumsum(mf):
        incl = jnp.dot(mf, tri, preferred_element_type=jnp.float32)
        row_tot = jnp.sum(mf, axis=1, keepdims=True)
        off = jnp.dot(strict, row_tot, preferred_element_type=jnp.float32)
        return incl - mf + off

    eqf = eq.astype(jnp.float32)
    eq_rank = excl_cumsum(eqf)
    keep = gt | (eq & (eq_rank < (kf32 - cnt_gt)))
    kpf = keep.astype(jnp.float32)
    kc = excl_cumsum(kpf)
    nidf = nid.astype(jnp.float32)
    pos = jnp.where(keep, kc, kf32 + (nidf - kc))
    p_ref[...] = pos.astype(jnp.int32)
    g_ref[...] = jnp.where(valid, f_ref[...] * attn, 0.0)


def _k4_body(p_hbm, g_hbm, x_hbm, out_hbm,
             p_v, ids_v, idx_v, gk_v, rows_v, kidx_sh, sem):
    cid = lax.axis_index("c")
    sid = lax.axis_index("s")
    pltpu.sync_copy(p_hbm.at[sid], p_v)
    def ib(i, _):
        ids_v[pl.ds(i * 16, 16)] = (sid * NODES_PER_SUB + i * 16
                                    + lax.broadcasted_iota(jnp.int32, (16,), 0))
        return 0
    lax.fori_loop(0, NODES_PER_SUB // 16, ib, 0)
    pltpu.sync_copy(ids_v, kidx_sh.at[p_v])
    plsc.subcore_barrier()
    wid = cid * NS + sid
    base = wid * ROWS_PER_SUB
    pltpu.sync_copy(kidx_sh.at[pl.ds(base, ROWS_PER_SUB)], idx_v)
    pltpu.async_copy(g_hbm.at[idx_v], gk_v, sem).wait()
    pltpu.async_copy(x_hbm.at[idx_v], rows_v, sem).wait()
    def rb(cchunk, _):
        gv = gk_v[pl.ds(cchunk * 16, 16)]
        for r in range(16):
            gr = gv[r]
            row = cchunk * 16 + r
            for l in range(8):
                sl = pl.ds(l * 16, 16)
                rows_v[row, sl] = rows_v[row, sl] * gr
        return 0
    lax.fori_loop(0, ROWS_PER_SUB // 16, rb, 0)
    pltpu.sync_copy(rows_v, out_hbm.at[pl.ds(base, ROWS_PER_SUB)])


def _sc_mesh():
    return plsc.VectorSubcoreMesh(core_axis_name="c", subcore_axis_name="s",
                                  num_cores=NC, num_subcores=NS)


@jax.jit
def kernel(input_feature, edge_index, edge_weight, weight, bias, w1, b):
    bias11 = bias.reshape(1, 1).astype(jnp.float32)
    w1a = jnp.asarray(w1, jnp.float32).reshape(1, 1)
    ba = jnp.asarray(b, jnp.float32).reshape(1, 1)
    norm_in = jnp.linalg.norm(input_feature, axis=-1, keepdims=True)

    KB = 10
    BR = N // KB
    s_col, norm_col, f_col = pl.pallas_call(
        _k1_body,
        grid=(KB,),
        in_specs=[
            pl.BlockSpec((BR, D), lambda i: (i, 0)),
            pl.BlockSpec((D, 1), lambda i: (0, 0)),
            pl.BlockSpec((1, 1), lambda i: (0, 0)),
            pl.BlockSpec((BR, 1), lambda i: (i, 0)),
        ],
        out_specs=[pl.BlockSpec((BR, 1), lambda i: (i, 0))] * 3,
        out_shape=[jax.ShapeDtypeStruct((N, 1), jnp.float32)] * 3,
    )(input_feature, weight, bias11, norm_in)

    src2 = edge_index[0].reshape(NW, EPT)
    dst2 = edge_index[1].reshape(NW, EPT)
    ew2 = edge_weight.reshape(NW, EPT)
    s_flat = s_col.reshape(N)
    norm_col = jnp.pad(norm_col, ((0, NPAD - N), (0, 0)))
    f_col = jnp.pad(f_col, ((0, NPAD - N), (0, 0)))

    k2 = pl.kernel(
        _k2_body,
        out_type=jax.ShapeDtypeStruct((NC, NPAD), jnp.float32),
        mesh=_sc_mesh(),
        scratch_types=[
            pltpu.VMEM((EPT,), jnp.int32),
            pltpu.VMEM((EPT,), jnp.int32),
            pltpu.VMEM((EPT,), jnp.float32),
            pltpu.VMEM((EPT,), jnp.float32),
            pltpu.VMEM((EPT,), jnp.float32),
            pltpu.VMEM_SHARED((NPAD,), jnp.float32),
            pltpu.SemaphoreType.DMA,
            pltpu.SemaphoreType.DMA,
            pltpu.SemaphoreType.DMA,
            pltpu.SemaphoreType.DMA,
            pltpu.SemaphoreType.DMA,
        ],
    )
    aggp = k2(src2, dst2, ew2, s_flat, jnp.zeros((NPAD,), jnp.float32))

    attn2d, p2d, g2d = pl.pallas_call(
        _k3_body,
        out_shape=[
            jax.ShapeDtypeStruct((NPAD // 128, 128), jnp.float32),
            jax.ShapeDtypeStruct((NPAD // 128, 128), jnp.int32),
            jax.ShapeDtypeStruct((NPAD // 128, 128), jnp.float32),
        ],
    )(aggp.reshape(2 * (NPAD // 128), 128),
      norm_col.reshape(NPAD // 128, 128),
      f_col.reshape(NPAD // 128, 128), w1a, ba)

    attn_score = attn2d.reshape(NPAD)[:N]
    p16 = p2d.reshape(NS, NODES_PER_SUB)
    g_flat = g2d.reshape(NPAD)

    k4 = pl.kernel(
        _k4_body,
        out_type=jax.ShapeDtypeStruct((ROWS_OUT, D), jnp.float32),
        mesh=_sc_mesh(),
        scratch_types=[
            pltpu.VMEM((NODES_PER_SUB,), jnp.int32),
            pltpu.VMEM((NODES_PER_SUB,), jnp.int32),
            pltpu.VMEM((ROWS_PER_SUB,), jnp.int32),
            pltpu.VMEM((ROWS_PER_SUB,), jnp.float32),
            pltpu.VMEM((ROWS_PER_SUB, D), jnp.float32),
            pltpu.VMEM_SHARED((NPAD,), jnp.int32),
            pltpu.SemaphoreType.DMA,
        ],
    )
    hidden_pad = k4(p16, g_flat, input_feature)
    hidden = hidden_pad[:KEEP]
    return hidden, attn_score


# speedup vs baseline: 16.2879x; 1.0173x over previous
import jax
import jax.numpy as jnp
from jax import lax
from jax.experimental import pallas as pl
from jax.experimental.pallas import tpu as pltpu
from jax.experimental.pallas import tpu_sc as plsc

C = 1.0
KEEP = 5000
N = 10000
NPAD = 10240
D = 128
E = 320000
MIN_NORM = 1e-15
EPS = 1e-5

NC = 2
NS = 16
NW = NC * NS
EPT = E // NW
NODES_PER_SUB = NPAD // NS
ROWS_OUT = 5120
ROWS_PER_SUB = ROWS_OUT // NW


def _k1_body(x_ref, wt_ref, bias_ref, norm_in_ref, s_ref, norm_ref, f_ref):
    x = x_ref[...]
    norm = norm_in_ref[...]
    ncl = jnp.maximum(norm, MIN_NORM)
    scaled = jnp.minimum(ncl, 1.0 - EPS)
    arct = 0.5 * (jnp.log1p(scaled) - jnp.log1p(-scaled))
    f = arct / ncl
    xt = f * x
    dot = jnp.dot(xt, wt_ref[...],
                  preferred_element_type=jnp.float32)
    hb = bias_ref[0, 0]
    hbn = jnp.maximum(jnp.sqrt(hb * hb), MIN_NORM)
    e = jnp.tanh(hbn) * hb / hbn
    maxn = 1.0 - EPS
    en = jnp.maximum(jnp.sqrt(e * e), MIN_NORM)
    h = jnp.where(en > maxn, e / en * maxn, e)
    x2 = dot * dot
    y2 = h * h
    xy = dot * h
    num = (1.0 + 2.0 * xy + y2) * dot + (1.0 - x2) * h
    den = 1.0 + 2.0 * xy + x2 * y2
    m = num / jnp.maximum(den, MIN_NORM)
    mn = jnp.maximum(jnp.sqrt(m * m), MIN_NORM)
    s = jnp.where(mn > maxn, m / mn * maxn, m)
    s_ref[...] = s
    norm_ref[...] = norm
    f_ref[...] = f


K2CH = 5
K2CE = EPT // K2CH


def _k2_body(src_hbm, dst_hbm, ew_hbm, s_hbm, zeros_hbm, agg_hbm,
             src_v, dst_v, ew_v, sval_v, val_v, agg_sh,
             sem_s, sem_d, sem_e, sem_g, sem_sc, sem_z):
    cid = lax.axis_index("c")
    sid = lax.axis_index("s")
    wid = cid * NS + sid
    c_src = pltpu.async_copy(src_hbm.at[wid], src_v, sem_s)
    c_dst = pltpu.async_copy(dst_hbm.at[wid], dst_v, sem_d)
    c_ew = pltpu.async_copy(ew_hbm.at[wid], ew_v, sem_e)
    c_z = pltpu.async_copy(zeros_hbm.at[pl.ds(sid * NODES_PER_SUB, NODES_PER_SUB)],
                           agg_sh.at[pl.ds(sid * NODES_PER_SUB, NODES_PER_SUB)],
                           sem_z)
    c_src.wait()

    def _gather(c):
        d = pltpu.make_async_copy(s_hbm.at[src_v.at[pl.ds(c * K2CE, K2CE)]],
                                  sval_v.at[pl.ds(c * K2CE, K2CE)], sem_g)
        d.start()
        return d

    g_cur = _gather(0)
    c_ew.wait()
    c_dst.wait()
    c_z.wait()
    plsc.subcore_barrier()
    scatters = []
    for c in range(K2CH):
        g_cur.wait()
        if c + 1 < K2CH:
            g_next = _gather(c + 1)
        def body(i, _, c=c):
            for u in range(5):
                sl = pl.ds(c * K2CE + i * 80 + u * 16, 16)
                val_v[sl] = sval_v[sl] * ew_v[sl]
            return 0
        lax.fori_loop(0, K2CE // 80, body, 0)
        d = pltpu.make_async_copy(
            val_v.at[pl.ds(c * K2CE, K2CE)],
            agg_sh.at[dst_v.at[pl.ds(c * K2CE, K2CE)]], sem_sc)
        d.start(add=True)
        scatters.append(d)
        if c + 1 < K2CH:
            g_cur = g_next
    for sc in scatters:
        sc.wait()
    plsc.subcore_barrier()
    pltpu.sync_copy(agg_sh.at[pl.ds(sid * NODES_PER_SUB, NODES_PER_SUB)],
                    agg_hbm.at[cid, pl.ds(sid * NODES_PER_SUB, NODES_PER_SUB)])


def _k3_body(agg_ref, norm_ref, f_ref, w1_ref, b_ref,
             attn_ref, p_ref, g_ref):
    R = NPAD // 128
    agg = agg_ref[0:R, :] + agg_ref[R:2 * R, :]
    un = jnp.maximum(jnp.sqrt(agg * agg), MIN_NORM)
    e0 = jnp.tanh(un) * agg / un
    maxn = 1.0 - EPS
    e0n = jnp.maximum(jnp.sqrt(e0 * e0), MIN_NORM)
    e0p = jnp.where(e0n > maxn, e0 / e0n * maxn, e0)
    at = jnp.tanh(e0p)
    w1 = w1_ref[0, 0]
    b = b_ref[0, 0]
    attn = w1 * at + (1.0 - w1) * (1.0 - norm_ref[...]) + b
    attn_ref[...] = attn
    rid = lax.broadcasted_iota(jnp.int32, (R, 128), 0)
    cidx = lax.broadcasted_iota(jnp.int32, (R, 128), 1)
    nid = rid * 128 + cidx
    valid = nid < N
    attn_eff = jnp.where(valid, attn, -jnp.inf)
    bu = lax.bitcast_convert_type(attn_eff, jnp.uint32)
    sign = bu >> jnp.uint32(31)
    flip = jnp.where(sign == jnp.uint32(1),
                     jnp.uint32(0xFFFFFFFF), jnp.uint32(0x80000000))
    u = bu ^ flip
    kf32 = jnp.float32(KEEP)
    def bit_body(j, t):
        cand = t | (jnp.uint32(1) << jnp.uint32(31 - j))
        cnt = jnp.sum(jnp.where(u >= cand, 1.0, 0.0))
        return jnp.where(cnt >= kf32, cand, t)
    thr = lax.fori_loop(0, 32, bit_body, jnp.uint32(0))
    gt = u > thr
    eq = u == thr
    cnt_gt = jnp.sum(jnp.where(gt, 1.0, 0.0))
    l_ids = lax.broadcasted_iota(jnp.int32, (128, 128), 0)
    c_ids = lax.broadcasted_iota(jnp.int32, (128, 128), 1)
    tri = (l_ids <= c_ids).astype(jnp.float32)
    r_ids = lax.broadcasted_iota(jnp.int32, (R, R), 0)
    q_ids = lax.broadcasted_iota(jnp.int32, (R, R), 1)
    strict = (q_ids < r_ids).astype(jnp.float32)

    def excl_cumsum(mf):
        incl = jnp.dot(mf, tri, preferred_element_type=jnp.float32)
        row_tot = jnp.sum(mf, axis=1, keepdims=True)
        off = jnp.dot(strict, row_tot, preferred_element_type=jnp.float32)
        return incl - mf + off

    eqf = eq.astype(jnp.float32)
    eq_rank = excl_cumsum(eqf)
    keep = gt | (eq & (eq_rank < (kf32 - cnt_gt)))
    kpf = keep.astype(jnp.float32)
    kc = excl_cumsum(kpf)
    nidf = nid.astype(jnp.float32)
    pos = jnp.where(keep, kc, kf32 + (nidf - kc))
    p_ref[...] = pos.astype(jnp.int32)
    g_ref[...] = jnp.where(valid, f_ref[...] * attn, 0.0)


def _k4_body(p_hbm, g_hbm, x_hbm, out_hbm,
             p_v, ids_v, idx_v, gk_v, rows_v, kidx_sh, sem):
    cid = lax.axis_index("c")
    sid = lax.axis_index("s")
    pltpu.sync_copy(p_hbm.at[sid], p_v)
    def ib(i, _):
        ids_v[pl.ds(i * 16, 16)] = (sid * NODES_PER_SUB + i * 16
                                    + lax.broadcasted_iota(jnp.int32, (16,), 0))
        return 0
    lax.fori_loop(0, NODES_PER_SUB // 16, ib, 0)
    pltpu.sync_copy(ids_v, kidx_sh.at[p_v])
    plsc.subcore_barrier()
    wid = cid * NS + sid
    base = wid * ROWS_PER_SUB
    pltpu.sync_copy(kidx_sh.at[pl.ds(base, ROWS_PER_SUB)], idx_v)
    pltpu.async_copy(g_hbm.at[idx_v], gk_v, sem).wait()
    pltpu.async_copy(x_hbm.at[idx_v], rows_v, sem).wait()
    def rb(cchunk, _):
        gv = gk_v[pl.ds(cchunk * 16, 16)]
        for r in range(16):
            gr = gv[r]
            row = cchunk * 16 + r
            for l in range(8):
                sl = pl.ds(l * 16, 16)
                rows_v[row, sl] = rows_v[row, sl] * gr
        return 0
    lax.fori_loop(0, ROWS_PER_SUB // 16, rb, 0)
    pltpu.sync_copy(rows_v, out_hbm.at[pl.ds(base, ROWS_PER_SUB)])


def _sc_mesh():
    return plsc.VectorSubcoreMesh(core_axis_name="c", subcore_axis_name="s",
                                  num_cores=NC, num_subcores=NS)


@jax.jit
def kernel(input_feature, edge_index, edge_weight, weight, bias, w1, b):
    bias11 = bias.reshape(1, 1).astype(jnp.float32)
    w1a = jnp.asarray(w1, jnp.float32).reshape(1, 1)
    ba = jnp.asarray(b, jnp.float32).reshape(1, 1)
    norm_in = jnp.linalg.norm(input_feature, axis=-1, keepdims=True)

    KB = 10
    BR = N // KB
    s_col, norm_col, f_col = pl.pallas_call(
        _k1_body,
        grid=(KB,),
        in_specs=[
            pl.BlockSpec((BR, D), lambda i: (i, 0)),
            pl.BlockSpec((D, 1), lambda i: (0, 0)),
            pl.BlockSpec((1, 1), lambda i: (0, 0)),
            pl.BlockSpec((BR, 1), lambda i: (i, 0)),
        ],
        out_specs=[pl.BlockSpec((BR, 1), lambda i: (i, 0))] * 3,
        out_shape=[jax.ShapeDtypeStruct((N, 1), jnp.float32)] * 3,
    )(input_feature, weight, bias11, norm_in)

    src2 = edge_index[0].reshape(NW, EPT)
    dst2 = edge_index[1].reshape(NW, EPT)
    ew2 = edge_weight.reshape(NW, EPT)
    s_flat = s_col.reshape(N)
    norm_col = jnp.pad(norm_col, ((0, NPAD - N), (0, 0)))
    f_col = jnp.pad(f_col, ((0, NPAD - N), (0, 0)))

    k2 = pl.kernel(
        _k2_body,
        out_type=jax.ShapeDtypeStruct((NC, NPAD), jnp.float32),
        mesh=_sc_mesh(),
        scratch_types=[
            pltpu.VMEM((EPT,), jnp.int32),
            pltpu.VMEM((EPT,), jnp.int32),
            pltpu.VMEM((EPT,), jnp.float32),
            pltpu.VMEM((EPT,), jnp.float32),
            pltpu.VMEM((EPT,), jnp.float32),
            pltpu.VMEM_SHARED((NPAD,), jnp.float32),
            pltpu.SemaphoreType.DMA,
            pltpu.SemaphoreType.DMA,
            pltpu.SemaphoreType.DMA,
            pltpu.SemaphoreType.DMA,
            pltpu.SemaphoreType.DMA,
            pltpu.SemaphoreType.DMA,
        ],
    )
    aggp = k2(src2, dst2, ew2, s_flat, jnp.zeros((NPAD,), jnp.float32))

    attn2d, p2d, g2d = pl.pallas_call(
        _k3_body,
        out_shape=[
            jax.ShapeDtypeStruct((NPAD // 128, 128), jnp.float32),
            jax.ShapeDtypeStruct((NPAD // 128, 128), jnp.int32),
            jax.ShapeDtypeStruct((NPAD // 128, 128), jnp.float32),
        ],
    )(aggp.reshape(2 * (NPAD // 128), 128),
      norm_col.reshape(NPAD // 128, 128),
      f_col.reshape(NPAD // 128, 128), w1a, ba)

    attn_score = attn2d.reshape(NPAD)[:N]
    p16 = p2d.reshape(NS, NODES_PER_SUB)
    g_flat = g2d.reshape(NPAD)

    k4 = pl.kernel(
        _k4_body,
        out_type=jax.ShapeDtypeStruct((ROWS_OUT, D), jnp.float32),
        mesh=_sc_mesh(),
        scratch_types=[
            pltpu.VMEM((NODES_PER_SUB,), jnp.int32),
            pltpu.VMEM((NODES_PER_SUB,), jnp.int32),
            pltpu.VMEM((ROWS_PER_SUB,), jnp.int32),
            pltpu.VMEM((ROWS_PER_SUB,), jnp.float32),
            pltpu.VMEM((ROWS_PER_SUB, D), jnp.float32),
            pltpu.VMEM_SHARED((NPAD,), jnp.int32),
            pltpu.SemaphoreType.DMA,
        ],
    )
    hidden_pad = k4(p16, g_flat, input_feature)
    hidden = hidden_pad[:KEEP]
    return hidden, attn_score


# speedup vs baseline: 17.6418x; 1.0831x over previous
import jax
import jax.numpy as jnp
from jax import lax
from jax.experimental import pallas as pl
from jax.experimental.pallas import tpu as pltpu
from jax.experimental.pallas import tpu_sc as plsc

C = 1.0
KEEP = 5000
N = 10000
NPAD = 10240
D = 128
E = 320000
MIN_NORM = 1e-15
EPS = 1e-5

NC = 2
NS = 16
NW = NC * NS
EPT = E // NW
NODES_PER_SUB = NPAD // NS
ROWS_OUT = 5120
ROWS_PER_SUB = ROWS_OUT // NW


def _k1_body(x_ref, wt_ref, bias_ref, norm_in_ref, s_ref, norm_ref, f_ref):
    x = x_ref[...]
    norm = norm_in_ref[...]
    ncl = jnp.maximum(norm, MIN_NORM)
    scaled = jnp.minimum(ncl, 1.0 - EPS)
    arct = 0.5 * (jnp.log1p(scaled) - jnp.log1p(-scaled))
    f = arct / ncl
    xt = f * x
    dot = jnp.dot(xt, wt_ref[...],
                  preferred_element_type=jnp.float32)
    hb = bias_ref[0, 0]
    hbn = jnp.maximum(jnp.sqrt(hb * hb), MIN_NORM)
    e = jnp.tanh(hbn) * hb / hbn
    maxn = 1.0 - EPS
    en = jnp.maximum(jnp.sqrt(e * e), MIN_NORM)
    h = jnp.where(en > maxn, e / en * maxn, e)
    x2 = dot * dot
    y2 = h * h
    xy = dot * h
    num = (1.0 + 2.0 * xy + y2) * dot + (1.0 - x2) * h
    den = 1.0 + 2.0 * xy + x2 * y2
    m = num / jnp.maximum(den, MIN_NORM)
    mn = jnp.maximum(jnp.sqrt(m * m), MIN_NORM)
    s = jnp.where(mn > maxn, m / mn * maxn, m)
    s_ref[...] = s
    norm_ref[...] = norm
    f_ref[...] = f


K2CH = 5
K2CE = EPT // K2CH


def _k2_body(src_hbm, dst_hbm, ew_hbm, s_hbm, zeros_hbm, agg_hbm,
             src_v, dst_v, ew_v, sval_v, val_v, agg_sh,
             sem_s, sem_d, sem_e, sem_g, sem_sc, sem_z):
    cid = lax.axis_index("c")
    sid = lax.axis_index("s")
    wid = cid * NS + sid
    ebase = wid * EPT
    c_src = pltpu.async_copy(src_hbm.at[pl.ds(ebase, EPT)], src_v, sem_s)
    c_dst = pltpu.async_copy(dst_hbm.at[pl.ds(ebase, EPT)], dst_v, sem_d)
    c_ew = pltpu.async_copy(ew_hbm.at[pl.ds(ebase, EPT)], ew_v, sem_e)
    c_z = pltpu.async_copy(zeros_hbm.at[pl.ds(sid * NODES_PER_SUB, NODES_PER_SUB)],
                           agg_sh.at[pl.ds(sid * NODES_PER_SUB, NODES_PER_SUB)],
                           sem_z)
    c_src.wait()

    def _gather(c):
        d = pltpu.make_async_copy(s_hbm.at[src_v.at[pl.ds(c * K2CE, K2CE)]],
                                  sval_v.at[pl.ds(c * K2CE, K2CE)], sem_g)
        d.start()
        return d

    g_cur = _gather(0)
    c_ew.wait()
    c_dst.wait()
    c_z.wait()
    plsc.subcore_barrier()
    scatters = []
    for c in range(K2CH):
        g_cur.wait()
        if c + 1 < K2CH:
            g_next = _gather(c + 1)
        def body(i, _, c=c):
            for u in range(5):
                sl = pl.ds(c * K2CE + i * 80 + u * 16, 16)
                val_v[sl] = sval_v[sl] * ew_v[sl]
            return 0
        lax.fori_loop(0, K2CE // 80, body, 0)
        d = pltpu.make_async_copy(
            val_v.at[pl.ds(c * K2CE, K2CE)],
            agg_sh.at[dst_v.at[pl.ds(c * K2CE, K2CE)]], sem_sc)
        d.start(add=True)
        scatters.append(d)
        if c + 1 < K2CH:
            g_cur = g_next
    for sc in scatters:
        sc.wait()
    plsc.subcore_barrier()
    pltpu.sync_copy(agg_sh.at[pl.ds(sid * NODES_PER_SUB, NODES_PER_SUB)],
                    agg_hbm.at[cid, pl.ds(sid * NODES_PER_SUB, NODES_PER_SUB)])


def _k3_body(agg_ref, norm_ref, f_ref, w1_ref, b_ref,
             attn_ref, p_ref, g_ref):
    R = NPAD // 128
    agg = agg_ref[0:R, :] + agg_ref[R:2 * R, :]
    un = jnp.maximum(jnp.sqrt(agg * agg), MIN_NORM)
    e0 = jnp.tanh(un) * agg / un
    maxn = 1.0 - EPS
    e0n = jnp.maximum(jnp.sqrt(e0 * e0), MIN_NORM)
    e0p = jnp.where(e0n > maxn, e0 / e0n * maxn, e0)
    at = jnp.tanh(e0p)
    w1 = w1_ref[0, 0]
    b = b_ref[0, 0]
    attn = w1 * at + (1.0 - w1) * (1.0 - norm_ref[...]) + b
    attn_ref[...] = attn
    rid = lax.broadcasted_iota(jnp.int32, (R, 128), 0)
    cidx = lax.broadcasted_iota(jnp.int32, (R, 128), 1)
    nid = rid * 128 + cidx
    valid = nid < N
    attn_eff = jnp.where(valid, attn, -jnp.inf)
    bu = lax.bitcast_convert_type(attn_eff, jnp.uint32)
    sign = bu >> jnp.uint32(31)
    flip = jnp.where(sign == jnp.uint32(1),
                     jnp.uint32(0xFFFFFFFF), jnp.uint32(0x80000000))
    u = bu ^ flip
    kf32 = jnp.float32(KEEP)
    def bit_body(j, t):
        cand = t | (jnp.uint32(1) << jnp.uint32(31 - j))
        cnt = jnp.sum(jnp.where(u >= cand, 1.0, 0.0))
        return jnp.where(cnt >= kf32, cand, t)
    thr = lax.fori_loop(0, 32, bit_body, jnp.uint32(0))
    gt = u > thr
    eq = u == thr
    cnt_gt = jnp.sum(jnp.where(gt, 1.0, 0.0))
    l_ids = lax.broadcasted_iota(jnp.int32, (128, 128), 0)
    c_ids = lax.broadcasted_iota(jnp.int32, (128, 128), 1)
    tri = (l_ids <= c_ids).astype(jnp.float32)
    r_ids = lax.broadcasted_iota(jnp.int32, (R, R), 0)
    q_ids = lax.broadcasted_iota(jnp.int32, (R, R), 1)
    strict = (q_ids < r_ids).astype(jnp.float32)

    def excl_cumsum(mf):
        incl = jnp.dot(mf, tri, preferred_element_type=jnp.float32)
        row_tot = jnp.sum(mf, axis=1, keepdims=True)
        off = jnp.dot(strict, row_tot, preferred_element_type=jnp.float32)
        return incl - mf + off

    eqf = eq.astype(jnp.float32)
    eq_rank = excl_cumsum(eqf)
    keep = gt | (eq & (eq_rank < (kf32 - cnt_gt)))
    kpf = keep.astype(jnp.float32)
    kc = excl_cumsum(kpf)
    nidf = nid.astype(jnp.float32)
    pos = jnp.where(keep, kc, kf32 + (nidf - kc))
    p_ref[...] = pos.astype(jnp.int32)
    g_ref[...] = jnp.where(valid, f_ref[...] * attn, 0.0)


def _k4_body(p_hbm, g_hbm, x_hbm, out_hbm,
             p_v, ids_v, idx_v, gk_v, rows_v, kidx_sh, sem_g, sem_x):
    cid = lax.axis_index("c")
    sid = lax.axis_index("s")
    pltpu.sync_copy(p_hbm.at[sid], p_v)
    def ib(i, _):
        ids_v[pl.ds(i * 16, 16)] = (sid * NODES_PER_SUB + i * 16
                                    + lax.broadcasted_iota(jnp.int32, (16,), 0))
        return 0
    lax.fori_loop(0, NODES_PER_SUB // 16, ib, 0)
    pltpu.sync_copy(ids_v, kidx_sh.at[p_v])
    plsc.subcore_barrier()
    wid = cid * NS + sid
    base = wid * ROWS_PER_SUB
    pltpu.sync_copy(kidx_sh.at[pl.ds(base, ROWS_PER_SUB)], idx_v)
    cg = pltpu.async_copy(g_hbm.at[idx_v], gk_v, sem_g)
    cx = pltpu.async_copy(x_hbm.at[idx_v], rows_v, sem_x)
    cg.wait()
    cx.wait()
    def rb(cchunk, _):
        gv = gk_v[pl.ds(cchunk * 16, 16)]
        for r in range(16):
            gr = gv[r]
            row = cchunk * 16 + r
            for l in range(8):
                sl = pl.ds(l * 16, 16)
                rows_v[row, sl] = rows_v[row, sl] * gr
        return 0
    lax.fori_loop(0, ROWS_PER_SUB // 16, rb, 0)
    @pl.when(wid < NW - 1)
    def _full():
        pltpu.sync_copy(rows_v, out_hbm.at[pl.ds(base, ROWS_PER_SUB)])
    @pl.when(wid == NW - 1)
    def _tail():
        pltpu.sync_copy(rows_v.at[0:KEEP - (NW - 1) * ROWS_PER_SUB],
                        out_hbm.at[pl.ds(base, KEEP - (NW - 1) * ROWS_PER_SUB)])


def _sc_mesh():
    return plsc.VectorSubcoreMesh(core_axis_name="c", subcore_axis_name="s",
                                  num_cores=NC, num_subcores=NS)


@jax.jit
def kernel(input_feature, edge_index, edge_weight, weight, bias, w1, b):
    bias11 = bias.reshape(1, 1).astype(jnp.float32)
    w1a = jnp.asarray(w1, jnp.float32).reshape(1, 1)
    ba = jnp.asarray(b, jnp.float32).reshape(1, 1)
    norm_in = jnp.linalg.norm(input_feature, axis=-1, keepdims=True)

    KB = 10
    BR = N // KB
    s_col, norm_col, f_col = pl.pallas_call(
        _k1_body,
        grid=(KB,),
        in_specs=[
            pl.BlockSpec((BR, D), lambda i: (i, 0)),
            pl.BlockSpec((D, 1), lambda i: (0, 0)),
            pl.BlockSpec((1, 1), lambda i: (0, 0)),
            pl.BlockSpec((BR, 1), lambda i: (i, 0)),
        ],
        out_specs=[pl.BlockSpec((BR, 1), lambda i: (i, 0))] * 3,
        out_shape=[jax.ShapeDtypeStruct((N, 1), jnp.float32)] * 3,
    )(input_feature, weight, bias11, norm_in)

    s_flat = s_col.reshape(N)
    norm_col = jnp.pad(norm_col, ((0, NPAD - N), (0, 0)))
    f_col = jnp.pad(f_col, ((0, NPAD - N), (0, 0)))

    k2 = pl.kernel(
        _k2_body,
        out_type=jax.ShapeDtypeStruct((NC, NPAD), jnp.float32),
        mesh=_sc_mesh(),
        scratch_types=[
            pltpu.VMEM((EPT,), jnp.int32),
            pltpu.VMEM((EPT,), jnp.int32),
            pltpu.VMEM((EPT,), jnp.float32),
            pltpu.VMEM((EPT,), jnp.float32),
            pltpu.VMEM((EPT,), jnp.float32),
            pltpu.VMEM_SHARED((NPAD,), jnp.float32),
            pltpu.SemaphoreType.DMA,
            pltpu.SemaphoreType.DMA,
            pltpu.SemaphoreType.DMA,
            pltpu.SemaphoreType.DMA,
            pltpu.SemaphoreType.DMA,
            pltpu.SemaphoreType.DMA,
        ],
    )
    aggp = k2(edge_index[0], edge_index[1], edge_weight, s_flat,
              jnp.zeros((NPAD,), jnp.float32))

    attn2d, p2d, g2d = pl.pallas_call(
        _k3_body,
        out_shape=[
            jax.ShapeDtypeStruct((NPAD // 128, 128), jnp.float32),
            jax.ShapeDtypeStruct((NPAD // 128, 128), jnp.int32),
            jax.ShapeDtypeStruct((NPAD // 128, 128), jnp.float32),
        ],
    )(aggp.reshape(2 * (NPAD // 128), 128),
      norm_col.reshape(NPAD // 128, 128),
      f_col.reshape(NPAD // 128, 128), w1a, ba)

    attn_score = attn2d.reshape(NPAD)[:N]
    p16 = p2d.reshape(NS, NODES_PER_SUB)
    g_flat = g2d.reshape(NPAD)

    k4 = pl.kernel(
        _k4_body,
        out_type=jax.ShapeDtypeStruct((KEEP, D), jnp.float32),
        mesh=_sc_mesh(),
        scratch_types=[
            pltpu.VMEM((NODES_PER_SUB,), jnp.int32),
            pltpu.VMEM((NODES_PER_SUB,), jnp.int32),
            pltpu.VMEM((ROWS_PER_SUB,), jnp.int32),
            pltpu.VMEM((ROWS_PER_SUB,), jnp.float32),
            pltpu.VMEM((ROWS_PER_SUB, D), jnp.float32),
            pltpu.VMEM_SHARED((NPAD,), jnp.int32),
            pltpu.SemaphoreType.DMA,
            pltpu.SemaphoreType.DMA,
        ],
    )
    hidden = k4(p16, g_flat, input_feature)
    return hidden, attn_score


# speedup vs baseline: 18.0197x; 1.0214x over previous
import jax
import jax.numpy as jnp
from jax import lax
from jax.experimental import pallas as pl
from jax.experimental.pallas import tpu as pltpu
from jax.experimental.pallas import tpu_sc as plsc

C = 1.0
KEEP = 5000
N = 10000
NPAD = 10240
D = 128
E = 320000
MIN_NORM = 1e-15
EPS = 1e-5

NC = 2
NS = 16
NW = NC * NS
EPT = E // NW
NODES_PER_SUB = NPAD // NS
ROWS_OUT = 5120
ROWS_PER_SUB = ROWS_OUT // NW


def _k1_body(x_ref, wt_ref, bias_ref, norm_in_ref, s_ref, norm_ref, f_ref):
    x = x_ref[...]
    norm = norm_in_ref[...]
    ncl = jnp.maximum(norm, MIN_NORM)
    scaled = jnp.minimum(ncl, 1.0 - EPS)
    arct = 0.5 * (jnp.log1p(scaled) - jnp.log1p(-scaled))
    f = arct / ncl
    xt = f * x
    dot = jnp.dot(xt, wt_ref[...],
                  preferred_element_type=jnp.float32)
    hb = bias_ref[0, 0]
    hbn = jnp.maximum(jnp.sqrt(hb * hb), MIN_NORM)
    e = jnp.tanh(hbn) * hb / hbn
    maxn = 1.0 - EPS
    en = jnp.maximum(jnp.sqrt(e * e), MIN_NORM)
    h = jnp.where(en > maxn, e / en * maxn, e)
    x2 = dot * dot
    y2 = h * h
    xy = dot * h
    num = (1.0 + 2.0 * xy + y2) * dot + (1.0 - x2) * h
    den = 1.0 + 2.0 * xy + x2 * y2
    m = num / jnp.maximum(den, MIN_NORM)
    mn = jnp.maximum(jnp.sqrt(m * m), MIN_NORM)
    s = jnp.where(mn > maxn, m / mn * maxn, m)
    s_ref[...] = s
    norm_ref[...] = norm
    f_ref[...] = f


K2CH = 5
K2CE = EPT // K2CH


def _k2_body(src_hbm, dst_hbm, ew_hbm, s_hbm, zeros_hbm, agg_hbm,
             src_v, dst_v, ew_v, sval_v, val_v, agg_sh,
             sem_s, sem_d, sem_e, sem_g, sem_sc, sem_z):
    cid = lax.axis_index("c")
    sid = lax.axis_index("s")
    wid = cid * NS + sid
    ebase = wid * EPT
    c_src = pltpu.async_copy(src_hbm.at[pl.ds(ebase, EPT)], src_v, sem_s)
    c_dst = pltpu.async_copy(dst_hbm.at[pl.ds(ebase, EPT)], dst_v, sem_d)
    c_ew = pltpu.async_copy(ew_hbm.at[pl.ds(ebase, EPT)], ew_v, sem_e)
    c_z = pltpu.async_copy(zeros_hbm.at[pl.ds(sid * NODES_PER_SUB, NODES_PER_SUB)],
                           agg_sh.at[pl.ds(sid * NODES_PER_SUB, NODES_PER_SUB)],
                           sem_z)
    c_src.wait()

    def _gather(c):
        d = pltpu.make_async_copy(s_hbm.at[src_v.at[pl.ds(c * K2CE, K2CE)]],
                                  sval_v.at[pl.ds(c * K2CE, K2CE)], sem_g)
        d.start()
        return d

    g_cur = _gather(0)
    c_ew.wait()
    c_dst.wait()
    c_z.wait()
    plsc.subcore_barrier()
    scatters = []
    for c in range(K2CH):
        g_cur.wait()
        if c + 1 < K2CH:
            g_next = _gather(c + 1)
        def body(i, _, c=c):
            for u in range(5):
                sl = pl.ds(c * K2CE + i * 80 + u * 16, 16)
                val_v[sl] = sval_v[sl] * ew_v[sl]
            return 0
        lax.fori_loop(0, K2CE // 80, body, 0)
        d = pltpu.make_async_copy(
            val_v.at[pl.ds(c * K2CE, K2CE)],
            agg_sh.at[dst_v.at[pl.ds(c * K2CE, K2CE)]], sem_sc)
        d.start(add=True)
        scatters.append(d)
        if c + 1 < K2CH:
            g_cur = g_next
    for sc in scatters:
        sc.wait()
    plsc.subcore_barrier()
    pltpu.sync_copy(agg_sh.at[pl.ds(sid * NODES_PER_SUB, NODES_PER_SUB)],
                    agg_hbm.at[cid, pl.ds(sid * NODES_PER_SUB, NODES_PER_SUB)])


def _k3_body(agg_ref, norm_ref, f_ref, w1_ref, b_ref,
             attn_ref, p_ref, g_ref):
    R = NPAD // 128
    agg = agg_ref[0:R, :] + agg_ref[R:2 * R, :]
    un = jnp.maximum(jnp.sqrt(agg * agg), MIN_NORM)
    e0 = jnp.tanh(un) * agg / un
    maxn = 1.0 - EPS
    e0n = jnp.maximum(jnp.sqrt(e0 * e0), MIN_NORM)
    e0p = jnp.where(e0n > maxn, e0 / e0n * maxn, e0)
    at = jnp.tanh(e0p)
    w1 = w1_ref[0, 0]
    b = b_ref[0, 0]
    attn = w1 * at + (1.0 - w1) * (1.0 - norm_ref[...]) + b
    attn_ref[...] = attn
    rid = lax.broadcasted_iota(jnp.int32, (R, 128), 0)
    cidx = lax.broadcasted_iota(jnp.int32, (R, 128), 1)
    nid = rid * 128 + cidx
    valid = nid < N
    attn_eff = jnp.where(valid, attn, -jnp.inf)
    bu = lax.bitcast_convert_type(attn_eff, jnp.uint32)
    sign = bu >> jnp.uint32(31)
    flip = jnp.where(sign == jnp.uint32(1),
                     jnp.uint32(0xFFFFFFFF), jnp.uint32(0x80000000))
    u = bu ^ flip
    kf32 = jnp.float32(KEEP)
    def bit_body(j, t):
        cand = t | (jnp.uint32(1) << jnp.uint32(31 - j))
        cnt = jnp.sum(jnp.where(u >= cand, 1.0, 0.0))
        return jnp.where(cnt >= kf32, cand, t)
    thr = lax.fori_loop(0, 32, bit_body, jnp.uint32(0))
    gt = u > thr
    eq = u == thr
    cnt_gt = jnp.sum(jnp.where(gt, 1.0, 0.0))
    l_ids = lax.broadcasted_iota(jnp.int32, (128, 128), 0)
    c_ids = lax.broadcasted_iota(jnp.int32, (128, 128), 1)
    tri = (l_ids <= c_ids).astype(jnp.float32)
    r_ids = lax.broadcasted_iota(jnp.int32, (R, R), 0)
    q_ids = lax.broadcasted_iota(jnp.int32, (R, R), 1)
    strict = (q_ids < r_ids).astype(jnp.float32)

    def excl_cumsum(mf):
        incl = jnp.dot(mf, tri, preferred_element_type=jnp.float32)
        row_tot = jnp.sum(mf, axis=1, keepdims=True)
        off = jnp.dot(strict, row_tot, preferred_element_type=jnp.float32)
        return incl - mf + off

    eqf = eq.astype(jnp.float32)
    eq_rank = excl_cumsum(eqf)
    keep = gt | (eq & (eq_rank < (kf32 - cnt_gt)))
    kpf = keep.astype(jnp.float32)
    kc = excl_cumsum(kpf)
    nidf = nid.astype(jnp.float32)
    pos = jnp.where(keep, kc, kf32 + (nidf - kc))
    p_ref[...] = pos.astype(jnp.int32)
    g_ref[...] = jnp.where(valid, f_ref[...] * attn, 0.0)


def _k4_body(p_hbm, g_hbm, x_hbm, out_hbm,
             p_v, ids_v, idx_v, gk_v, rows_v, kidx_sh, sem_g, sem_x):
    cid = lax.axis_index("c")
    sid = lax.axis_index("s")
    pltpu.sync_copy(p_hbm.at[sid], p_v)
    def ib(i, _):
        ids_v[pl.ds(i * 16, 16)] = (sid * NODES_PER_SUB + i * 16
                                    + lax.broadcasted_iota(jnp.int32, (16,), 0))
        return 0
    lax.fori_loop(0, NODES_PER_SUB // 16, ib, 0)
    pltpu.sync_copy(ids_v, kidx_sh.at[p_v])
    plsc.subcore_barrier()
    wid = cid * NS + sid
    base = wid * ROWS_PER_SUB
    pltpu.sync_copy(kidx_sh.at[pl.ds(base, ROWS_PER_SUB)], idx_v)
    cg = pltpu.async_copy(g_hbm.at[idx_v], gk_v, sem_g)
    cx = pltpu.async_copy(x_hbm.at[idx_v], rows_v, sem_x)
    cg.wait()
    cx.wait()
    def rb(cchunk, _):
        gv = gk_v[pl.ds(cchunk * 16, 16)]
        for r in range(16):
            gr = gv[r]
            row = cchunk * 16 + r
            for l in range(8):
                sl = pl.ds(l * 16, 16)
                rows_v[row, sl] = rows_v[row, sl] * gr
        return 0
    lax.fori_loop(0, ROWS_PER_SUB // 16, rb, 0)
    @pl.when(wid < NW - 1)
    def _full():
        pltpu.sync_copy(rows_v, out_hbm.at[pl.ds(base, ROWS_PER_SUB)])
    @pl.when(wid == NW - 1)
    def _tail():
        pltpu.sync_copy(rows_v.at[0:KEEP - (NW - 1) * ROWS_PER_SUB],
                        out_hbm.at[pl.ds(base, KEEP - (NW - 1) * ROWS_PER_SUB)])


def _sc_mesh():
    return plsc.VectorSubcoreMesh(core_axis_name="c", subcore_axis_name="s",
                                  num_cores=NC, num_subcores=NS)


@jax.jit
def kernel(input_feature, edge_index, edge_weight, weight, bias, w1, b):
    bias11 = bias.reshape(1, 1).astype(jnp.float32)
    w1a = jnp.asarray(w1, jnp.float32).reshape(1, 1)
    ba = jnp.asarray(b, jnp.float32).reshape(1, 1)
    norm_in = jnp.linalg.norm(input_feature, axis=-1, keepdims=True)

    KB = 5
    BR = N // KB
    s_col, norm_col, f_col = pl.pallas_call(
        _k1_body,
        grid=(KB,),
        in_specs=[
            pl.BlockSpec((BR, D), lambda i: (i, 0)),
            pl.BlockSpec((D, 1), lambda i: (0, 0)),
            pl.BlockSpec((1, 1), lambda i: (0, 0)),
            pl.BlockSpec((BR, 1), lambda i: (i, 0)),
        ],
        out_specs=[pl.BlockSpec((BR, 1), lambda i: (i, 0))] * 3,
        out_shape=[jax.ShapeDtypeStruct((N, 1), jnp.float32)] * 3,
    )(input_feature, weight, bias11, norm_in)

    s_flat = s_col.reshape(N)
    norm_col = jnp.pad(norm_col, ((0, NPAD - N), (0, 0)))
    f_col = jnp.pad(f_col, ((0, NPAD - N), (0, 0)))

    k2 = pl.kernel(
        _k2_body,
        out_type=jax.ShapeDtypeStruct((NC, NPAD), jnp.float32),
        mesh=_sc_mesh(),
        scratch_types=[
            pltpu.VMEM((EPT,), jnp.int32),
            pltpu.VMEM((EPT,), jnp.int32),
            pltpu.VMEM((EPT,), jnp.float32),
            pltpu.VMEM((EPT,), jnp.float32),
            pltpu.VMEM((EPT,), jnp.float32),
            pltpu.VMEM_SHARED((NPAD,), jnp.float32),
            pltpu.SemaphoreType.DMA,
            pltpu.SemaphoreType.DMA,
            pltpu.SemaphoreType.DMA,
            pltpu.SemaphoreType.DMA,
            pltpu.SemaphoreType.DMA,
            pltpu.SemaphoreType.DMA,
        ],
    )
    aggp = k2(edge_index[0], edge_index[1], edge_weight, s_flat,
              jnp.zeros((NPAD,), jnp.float32))

    attn2d, p2d, g2d = pl.pallas_call(
        _k3_body,
        out_shape=[
            jax.ShapeDtypeStruct((NPAD // 128, 128), jnp.float32),
            jax.ShapeDtypeStruct((NPAD // 128, 128), jnp.int32),
            jax.ShapeDtypeStruct((NPAD // 128, 128), jnp.float32),
        ],
    )(aggp.reshape(2 * (NPAD // 128), 128),
      norm_col.reshape(NPAD // 128, 128),
      f_col.reshape(NPAD // 128, 128), w1a, ba)

    attn_score = attn2d.reshape(NPAD)[:N]
    p16 = p2d.reshape(NS, NODES_PER_SUB)
    g_flat = g2d.reshape(NPAD)

    k4 = pl.kernel(
        _k4_body,
        out_type=jax.ShapeDtypeStruct((KEEP, D), jnp.float32),
        mesh=_sc_mesh(),
        scratch_types=[
            pltpu.VMEM((NODES_PER_SUB,), jnp.int32),
            pltpu.VMEM((NODES_PER_SUB,), jnp.int32),
            pltpu.VMEM((ROWS_PER_SUB,), jnp.int32),
            pltpu.VMEM((ROWS_PER_SUB,), jnp.float32),
            pltpu.VMEM((ROWS_PER_SUB, D), jnp.float32),
            pltpu.VMEM_SHARED((NPAD,), jnp.int32),
            pltpu.SemaphoreType.DMA,
            pltpu.SemaphoreType.DMA,
        ],
    )
    hidden = k4(p16, g_flat, input_feature)
    return hidden, attn_score
